# baseline (speedup 1.0000x reference)
import functools

import jax
import jax.numpy as jnp
from jax import lax
from jax.experimental import pallas as pl
from jax.experimental.pallas import tpu as pltpu
from jax.experimental.pallas import tpu_sc as plsc

_N = 10000
_E = 320000
_NG = 32
_NSUB = 16
_NCORE = 2
_CHUNK = 128
_NCHUNK = 160
_EPAD = _NSUB * _NCHUNK * _CHUNK
_NACC = 10112


def _make_edge_agg(dh2, nbuf, sp_table=False):
    mesh = plsc.VectorSubcoreMesh(core_axis_name="c", subcore_axis_name="s")
    zrows = _NACC // _NSUB
    nq = _NCHUNK // nbuf

    @functools.partial(
        pl.kernel,
        out_type=jax.ShapeDtypeStruct((_NCORE, _NACC, dh2), jnp.float32),
        mesh=mesh,
        scratch_types=[
            pltpu.VMEM((_NCHUNK, _CHUNK), jnp.int32),
            pltpu.VMEM((_NCHUNK, _CHUNK), jnp.int32),
            pltpu.VMEM((nbuf, _CHUNK, dh2), jnp.float32),
            pltpu.VMEM_SHARED((_NACC, dh2), jnp.float32),
            pltpu.VMEM_SHARED((_NACC, dh2), jnp.float32) if sp_table else None,
            [pltpu.SemaphoreType.DMA] * nbuf,
            [pltpu.SemaphoreType.DMA] * nbuf,
        ],
        compiler_params=pltpu.CompilerParams(use_tc_tiling_on_sc=False),
    )
    def agg(h_hbm, src_hbm, dst_hbm, zero_hbm, out_hbm,
            src_v, dst_v, rows_v, acc, h_s, gsems, ssems):
        cid = lax.axis_index("c")
        sid = lax.axis_index("s")
        hview = h_hbm.at[cid]
        pltpu.sync_copy(src_hbm.at[sid], src_v)
        if sp_table:
            pltpu.sync_copy(hview.at[pl.ds(sid * zrows, zrows)],
                            h_s.at[pl.ds(sid * zrows, zrows)])
        pltpu.sync_copy(dst_hbm.at[sid], dst_v)
        pltpu.sync_copy(zero_hbm.at[pl.ds(sid * zrows, zrows)],
                        acc.at[pl.ds(sid * zrows, zrows)])
        if sp_table:
            plsc.subcore_barrier()
            h_c = h_s
        else:
            h_c = hview
        for b in range(nbuf):
            pltpu.async_copy(h_c.at[src_v.at[b]], rows_v.at[b], gsems[b])
        plsc.subcore_barrier()

        def body(q, carry):
            j0 = nbuf * q
            for b in range(nbuf):
                jb = j0 + b
                pltpu.make_async_copy(
                    h_c.at[src_v.at[jb]], rows_v.at[b], gsems[b]).wait()
                pltpu.async_copy(
                    rows_v.at[b], acc.at[dst_v.at[jb]], ssems[b], add=True)

            @pl.when(q + 1 < nq)
            def _():
                for b in range(nbuf):
                    jb = j0 + b
                    pltpu.make_async_copy(
                        rows_v.at[b], acc.at[dst_v.at[jb]], ssems[b]).wait()
                    pltpu.async_copy(
                        h_c.at[src_v.at[jb + nbuf]], rows_v.at[b], gsems[b])

            @pl.when(q + 1 >= nq)
            def _():
                for b in range(nbuf):
                    jb = j0 + b
                    pltpu.make_async_copy(
                        rows_v.at[b], acc.at[dst_v.at[jb]], ssems[b]).wait()

            return carry

        lax.fori_loop(0, nq, body, 0)
        plsc.subcore_barrier()
        pltpu.sync_copy(acc.at[pl.ds(sid * zrows, zrows)],
                        out_hbm.at[cid].at[pl.ds(sid * zrows, zrows)])

    return agg


_agg64 = _make_edge_agg(64, 5)
_agg32 = _make_edge_agg(32, 10, sp_table=True)
_agg16 = _make_edge_agg(16, 10, sp_table=True)


def _mm_relu(a, W, b, blk=2000):
    p, n, dinp = a.shape
    dout = W.shape[1]
    dh2 = dout // 2

    def body(a_ref, w_ref, b_ref, o_ref):
        w = w_ref[...]
        h = jnp.dot(a_ref[0], w[:dinp], preferred_element_type=jnp.float32)
        for k in range(1, p):
            h = h + jnp.dot(a_ref[k], w[k * dinp:(k + 1) * dinp],
                            preferred_element_type=jnp.float32)
        h = jnp.maximum(h + b_ref[...], 0.0)
        o_ref[0] = h[:, :dh2]
        o_ref[1] = h[:, dh2:]

    return pl.pallas_call(
        body,
        grid=(n // blk,),
        in_specs=[pl.BlockSpec((p, blk, dinp), lambda r: (0, r, 0)),
                  pl.BlockSpec(W.shape, lambda r: (0, 0)),
                  pl.BlockSpec((1, dout), lambda r: (0, 0))],
        out_specs=pl.BlockSpec((2, blk, dh2), lambda r: (0, r, 0)),
        out_shape=jax.ShapeDtypeStruct((2, n, dh2), jnp.float32),
    )(a, W, b.reshape(1, dout))


def _pool_head(a, i2, Wd, bd):
    dh2 = a.shape[2]

    def body(a_ref, i_ref, wd_ref, bd_ref, o_ref):
        iv = i_ref[...]
        gcol = lax.broadcasted_iota(jnp.int32, (_N, _NG), 1)
        e = (iv == gcol).astype(jnp.float32)
        dn = (((0,), (0,)), ((), ()))
        a0 = a_ref[0][: _N]
        a1 = a_ref[1][: _N]
        s0 = lax.dot_general(e, a0, dn, preferred_element_type=jnp.float32)
        s1 = lax.dot_general(e, a1, dn, preferred_element_type=jnp.float32)
        cnt = jnp.maximum(jnp.sum(e, axis=0), 1.0)[:, None]
        wd = wd_ref[...]
        logits = (jnp.dot(s0 / cnt, wd[:dh2], preferred_element_type=jnp.float32)
                  + jnp.dot(s1 / cnt, wd[dh2:], preferred_element_type=jnp.float32)
                  + bd_ref[...])
        m = jnp.max(logits, axis=1, keepdims=True)
        ex = jnp.exp(logits - m)
        o_ref[...] = ex / jnp.sum(ex, axis=1, keepdims=True)

    return pl.pallas_call(
        body,
        out_shape=jax.ShapeDtypeStruct((_NG, Wd.shape[1]), jnp.float32),
    )(a, i2, Wd, bd)


def kernel(x, edge_index, i, W1, b1, W2, b2, W3, b3, Wd, bd):
    pad = _EPAD - _E
    padv = jax.lax.iota(jnp.int32, pad)
    src = jnp.concatenate(
        [edge_index[0], padv % _N]).reshape(_NSUB, _NCHUNK, _CHUNK)
    dst = jnp.concatenate(
        [edge_index[1], _N + padv % (_NACC - _N)]).reshape(_NSUB, _NCHUNK, _CHUNK)

    h1 = _mm_relu(x[None], W1, b1, blk=2000)
    a1 = _agg64(h1, src, dst, jnp.zeros((_NACC, 64), jnp.float32))
    h2 = _mm_relu(a1, W2, b2, blk=2528)
    a2 = _agg32(h2, src, dst, jnp.zeros((_NACC, 32), jnp.float32))
    h3 = _mm_relu(a2, W3, b3, blk=2528)
    a3 = _agg16(h3, src, dst, jnp.zeros((_NACC, 16), jnp.float32))
    return _pool_head(a3, i.reshape(_N, 1), Wd, bd.reshape(1, -1))


# speedup vs baseline: 1.1125x; 1.1125x over previous
import functools

import jax
import jax.numpy as jnp
from jax import lax
from jax.experimental import pallas as pl
from jax.experimental.pallas import tpu as pltpu
from jax.experimental.pallas import tpu_sc as plsc

_N = 10000
_E = 320000
_NG = 32
_NSUB = 16
_NCORE = 2
_CHUNK = 128
_NCHUNK = 160
_EPAD = _NSUB * _NCHUNK * _CHUNK
_NACC = 10112


def _make_edge_agg(dh2, nbuf, sp_table=False):
    mesh = plsc.VectorSubcoreMesh(core_axis_name="c", subcore_axis_name="s")
    zrows = _NACC // _NSUB
    nq = _NCHUNK // nbuf

    @functools.partial(
        pl.kernel,
        out_type=jax.ShapeDtypeStruct((_NCORE, _NACC, dh2), jnp.float32),
        mesh=mesh,
        scratch_types=[
            pltpu.VMEM((_NCHUNK, _CHUNK), jnp.int32),
            pltpu.VMEM((_NCHUNK, _CHUNK), jnp.int32),
            pltpu.VMEM((nbuf, _CHUNK, dh2), jnp.float32),
            pltpu.VMEM_SHARED((_NACC, dh2), jnp.float32),
            pltpu.VMEM_SHARED((_NACC, dh2), jnp.float32) if sp_table else None,
            [pltpu.SemaphoreType.DMA] * nbuf,
            [pltpu.SemaphoreType.DMA] * nbuf,
        ],
        compiler_params=pltpu.CompilerParams(use_tc_tiling_on_sc=False),
    )
    def agg(h_hbm, src_hbm, dst_hbm, zero_hbm, out_hbm,
            src_v, dst_v, rows_v, acc, h_s, gsems, ssems):
        cid = lax.axis_index("c")
        sid = lax.axis_index("s")
        hview = h_hbm.at[cid]
        pltpu.sync_copy(src_hbm.at[sid], src_v)
        if sp_table:
            pltpu.sync_copy(hview.at[pl.ds(sid * zrows, zrows)],
                            h_s.at[pl.ds(sid * zrows, zrows)])
        pltpu.sync_copy(dst_hbm.at[sid], dst_v)
        pltpu.sync_copy(zero_hbm.at[pl.ds(sid * zrows, zrows)],
                        acc.at[pl.ds(sid * zrows, zrows)])
        if sp_table:
            plsc.subcore_barrier()
            h_c = h_s
        else:
            h_c = hview
        for b in range(nbuf):
            pltpu.async_copy(h_c.at[src_v.at[b]], rows_v.at[b], gsems[b])
        plsc.subcore_barrier()

        def body(q, carry):
            j0 = nbuf * q
            for b in range(nbuf):
                jb = j0 + b
                pltpu.make_async_copy(
                    h_c.at[src_v.at[jb]], rows_v.at[b], gsems[b]).wait()
                pltpu.async_copy(
                    rows_v.at[b], acc.at[dst_v.at[jb]], ssems[b], add=True)

            @pl.when(q + 1 < nq)
            def _():
                for b in range(nbuf):
                    jb = j0 + b
                    pltpu.make_async_copy(
                        rows_v.at[b], acc.at[dst_v.at[jb]], ssems[b]).wait()
                    pltpu.async_copy(
                        h_c.at[src_v.at[jb + nbuf]], rows_v.at[b], gsems[b])

            @pl.when(q + 1 >= nq)
            def _():
                for b in range(nbuf):
                    jb = j0 + b
                    pltpu.make_async_copy(
                        rows_v.at[b], acc.at[dst_v.at[jb]], ssems[b]).wait()

            return carry

        lax.fori_loop(0, nq, body, 0)
        plsc.subcore_barrier()
        pltpu.sync_copy(acc.at[pl.ds(sid * zrows, zrows)],
                        out_hbm.at[cid].at[pl.ds(sid * zrows, zrows)])

    return agg


def _make_edge_agg_pool(dh2, nbuf):
    mesh = plsc.VectorSubcoreMesh(core_axis_name="c", subcore_axis_name="s")
    zrows = _NACC // _NSUB
    nq = _NCHUNK // nbuf
    nseg = (zrows + 8) // _CHUNK

    @functools.partial(
        pl.kernel,
        out_type=[jax.ShapeDtypeStruct((_NCORE, _NG, dh2), jnp.float32),
                  jax.ShapeDtypeStruct((_NCORE, _NG, dh2), jnp.float32)],
        mesh=mesh,
        scratch_types=[
            pltpu.VMEM((_NCHUNK, _CHUNK), jnp.int32),
            pltpu.VMEM((_NCHUNK, _CHUNK), jnp.int32),
            pltpu.VMEM((nbuf, _CHUNK, dh2), jnp.float32),
            pltpu.VMEM((nseg * _CHUNK, dh2), jnp.float32),
            pltpu.VMEM((nseg, _CHUNK), jnp.int32),
            pltpu.VMEM((_CHUNK, dh2), jnp.float32),
            pltpu.VMEM_SHARED((_NACC, dh2), jnp.float32),
            pltpu.VMEM_SHARED((40, dh2), jnp.float32),
            pltpu.VMEM_SHARED((40, dh2), jnp.float32),
            [pltpu.SemaphoreType.DMA] * nbuf,
            [pltpu.SemaphoreType.DMA] * nbuf,
        ],
        compiler_params=pltpu.CompilerParams(use_tc_tiling_on_sc=False),
    )
    def agg(h_hbm, src_hbm, dst_hbm, zero_hbm, iseg_hbm, ones_hbm,
            outs_hbm, outc_hbm,
            src_v, dst_v, rows_v, tbuf, ibuf, obuf, acc, pooled, pcnt,
            gsems, ssems):
        cid = lax.axis_index("c")
        sid = lax.axis_index("s")
        h_c = h_hbm.at[cid]
        pltpu.sync_copy(src_hbm.at[sid], src_v)
        for b in range(nbuf):
            pltpu.async_copy(h_c.at[src_v.at[b]], rows_v.at[b], gsems[b])
        pltpu.sync_copy(dst_hbm.at[sid], dst_v)
        pltpu.sync_copy(zero_hbm.at[pl.ds(sid * zrows, zrows)],
                        acc.at[pl.ds(sid * zrows, zrows)])
        pltpu.sync_copy(iseg_hbm.at[sid], ibuf)
        pltpu.sync_copy(ones_hbm, obuf)

        @pl.when(sid == 0)
        def _():
            pltpu.sync_copy(zero_hbm.at[pl.ds(0, 40)], pooled)
            pltpu.sync_copy(zero_hbm.at[pl.ds(40, 40)], pcnt)

        plsc.subcore_barrier()

        def body(q, carry):
            j0 = nbuf * q
            for b in range(nbuf):
                jb = j0 + b
                pltpu.make_async_copy(
                    h_c.at[src_v.at[jb]], rows_v.at[b], gsems[b]).wait()
                pltpu.async_copy(
                    rows_v.at[b], acc.at[dst_v.at[jb]], ssems[b], add=True)

            @pl.when(q + 1 < nq)
            def _():
                for b in range(nbuf):
                    jb = j0 + b
                    pltpu.make_async_copy(
                        rows_v.at[b], acc.at[dst_v.at[jb]], ssems[b]).wait()
                    pltpu.async_copy(
                        h_c.at[src_v.at[jb + nbuf]], rows_v.at[b], gsems[b])

            @pl.when(q + 1 >= nq)
            def _():
                for b in range(nbuf):
                    jb = j0 + b
                    pltpu.make_async_copy(
                        rows_v.at[b], acc.at[dst_v.at[jb]], ssems[b]).wait()

            return carry

        lax.fori_loop(0, nq, body, 0)
        plsc.subcore_barrier()

        pltpu.sync_copy(acc.at[pl.ds(sid * zrows, zrows)],
                        tbuf.at[pl.ds(0, zrows)])
        pltpu.sync_copy(acc.at[pl.ds(0, nseg * _CHUNK - zrows)],
                        tbuf.at[pl.ds(zrows, nseg * _CHUNK - zrows)])
        for c in range(nseg):
            pltpu.async_copy(tbuf.at[pl.ds(c * _CHUNK, _CHUNK)],
                             pooled.at[ibuf.at[c]], gsems[c], add=True)
            pltpu.async_copy(obuf, pcnt.at[ibuf.at[c]], ssems[c], add=True)
        for c in range(nseg):
            pltpu.make_async_copy(tbuf.at[pl.ds(c * _CHUNK, _CHUNK)],
                                  pooled.at[ibuf.at[c]], gsems[c]).wait()
            pltpu.make_async_copy(obuf, pcnt.at[ibuf.at[c]], ssems[c]).wait()
        plsc.subcore_barrier()

        @pl.when(sid == 0)
        def _():
            pltpu.sync_copy(pooled.at[pl.ds(0, _NG)], outs_hbm.at[cid])
            pltpu.sync_copy(pcnt.at[pl.ds(0, _NG)], outc_hbm.at[cid])

    return agg


_agg64 = _make_edge_agg(64, 5)
_agg32 = _make_edge_agg(32, 10)
_agg16p = _make_edge_agg_pool(16, 10)


def _mm_relu(a, W, b, blk=2000):
    p, n, dinp = a.shape
    dout = W.shape[1]
    dh2 = dout // 2

    def body(a_ref, w_ref, b_ref, o_ref):
        w = w_ref[...]
        h = jnp.dot(a_ref[0], w[:dinp], preferred_element_type=jnp.float32)
        for k in range(1, p):
            h = h + jnp.dot(a_ref[k], w[k * dinp:(k + 1) * dinp],
                            preferred_element_type=jnp.float32)
        h = jnp.maximum(h + b_ref[...], 0.0)
        o_ref[0] = h[:, :dh2]
        o_ref[1] = h[:, dh2:]

    return pl.pallas_call(
        body,
        grid=(n // blk,),
        in_specs=[pl.BlockSpec((p, blk, dinp), lambda r: (0, r, 0)),
                  pl.BlockSpec(W.shape, lambda r: (0, 0)),
                  pl.BlockSpec((1, dout), lambda r: (0, 0))],
        out_specs=pl.BlockSpec((2, blk, dh2), lambda r: (0, r, 0)),
        out_shape=jax.ShapeDtypeStruct((2, n, dh2), jnp.float32),
    )(a, W, b.reshape(1, dout))


def _head(ps, pc, Wd, bd):
    dh2 = ps.shape[2]

    def body(ps_ref, pc_ref, wd_ref, bd_ref, o_ref):
        cnt = jnp.maximum(pc_ref[0][:, :1], 1.0)
        wd = wd_ref[...]
        logits = (jnp.dot(ps_ref[0] / cnt, wd[:dh2],
                          preferred_element_type=jnp.float32)
                  + jnp.dot(ps_ref[1] / cnt, wd[dh2:],
                            preferred_element_type=jnp.float32)
                  + bd_ref[...])
        m = jnp.max(logits, axis=1, keepdims=True)
        ex = jnp.exp(logits - m)
        o_ref[...] = ex / jnp.sum(ex, axis=1, keepdims=True)

    return pl.pallas_call(
        body,
        out_shape=jax.ShapeDtypeStruct((_NG, Wd.shape[1]), jnp.float32),
    )(ps, pc, Wd, bd)


def kernel(x, edge_index, i, W1, b1, W2, b2, W3, b3, Wd, bd):
    pad = _EPAD - _E
    padv = jax.lax.iota(jnp.int32, pad)
    src = jnp.concatenate(
        [edge_index[0], padv % _N]).reshape(_NSUB, _NCHUNK, _CHUNK)
    dst = jnp.concatenate(
        [edge_index[1], _N + padv % (_NACC - _N)]).reshape(_NSUB, _NCHUNK, _CHUNK)

    h1 = _mm_relu(x[None], W1, b1, blk=2000)
    a1 = _agg64(h1, src, dst, jnp.zeros((_NACC, 64), jnp.float32))
    h2 = _mm_relu(a1, W2, b2, blk=2528)
    a2 = _agg32(h2, src, dst, jnp.zeros((_NACC, 32), jnp.float32))
    h3 = _mm_relu(a2, W3, b3, blk=2528)

    ii = jnp.concatenate([i, jnp.full((_NACC - _N,), _NG, jnp.int32)])
    ii = ii.reshape(_NSUB, _NACC // _NSUB)
    ii = jnp.concatenate([ii, jnp.full((_NSUB, 8), _NG, jnp.int32)], axis=1)
    iseg = ii.reshape(_NSUB, 5, _CHUNK)

    ps, pc = _agg16p(h3, src, dst, jnp.zeros((_NACC, 16), jnp.float32),
                     iseg, jnp.ones((_CHUNK, 16), jnp.float32))
    return _head(ps, pc, Wd, bd.reshape(1, -1))
